# Initial kernel scaffold; baseline (speedup 1.0000x reference)
import jax
import jax.numpy as jnp
from jax.experimental import pallas as pl


def kernel(x, Wa, ba, Wb, bb):
    raise NotImplementedError("write your pallas kernel here")


# speedup vs baseline: 4.3106x; 4.3106x over previous
import functools

import jax
import jax.numpy as jnp
from jax.experimental import pallas as pl

GRID = 8
NMS_SIZE = 4
DET_THRESH = 0.015
TOP_K = 300
K_CAND = 1024
H = W = 64
HW = H * W
CIN = 128
CMID = 256
COUT = 65
HEAT = H * GRID


def _head_body(xp_ref, wa_ref, ba_ref, wb_ref, bb_ref, logits_ref, prob_ref):
    acc = jnp.zeros((HW, CMID), jnp.float32)
    for t in range(9):
        dy, dx = t // 3, t % 3
        patch = xp_ref[0, dy:dy + H, dx:dx + W, :].reshape(HW, CIN)
        acc = acc + jax.lax.dot(patch, wa_ref[t],
                                preferred_element_type=jnp.float32)
    h = jnp.maximum(acc + ba_ref[...], 0.0)
    lg = jax.lax.dot(h, wb_ref[...],
                     preferred_element_type=jnp.float32) + bb_ref[...]
    logits_ref[0] = lg
    m = jnp.max(lg, axis=1, keepdims=True)
    e = jnp.exp(lg - m)
    s = jnp.sum(e, axis=1, keepdims=True)
    prob_ref[0] = (e / s)[:, :GRID * GRID]


def _conv_softmax(xp, wa, ba, wb, bb):
    B = xp.shape[0]
    return pl.pallas_call(
        _head_body,
        grid=(B,),
        in_specs=[
            pl.BlockSpec((1, H + 8, W + 8, CIN), lambda b: (b, 0, 0, 0)),
            pl.BlockSpec((9, CIN, CMID), lambda b: (0, 0, 0)),
            pl.BlockSpec((1, CMID), lambda b: (0, 0)),
            pl.BlockSpec((CMID, COUT), lambda b: (0, 0)),
            pl.BlockSpec((1, COUT), lambda b: (0, 0)),
        ],
        out_specs=[
            pl.BlockSpec((1, HW, COUT), lambda b: (b, 0, 0)),
            pl.BlockSpec((1, HW, GRID * GRID), lambda b: (b, 0, 0)),
        ],
        out_shape=[
            jax.ShapeDtypeStruct((B, HW, COUT), jnp.float32),
            jax.ShapeDtypeStruct((B, HW, GRID * GRID), jnp.float32),
        ],
    )(xp, wa, ba, wb, bb)


def _nms_tail(heat_flat):
    B = heat_flat.shape[0]
    scores, idx = jax.lax.top_k(heat_flat, K_CAND)
    ys = (idx // HEAT).astype(jnp.float32)
    xs = (idx % HEAT).astype(jnp.float32)
    dy = jnp.abs(ys[:, :, None] - ys[:, None, :])
    dx = jnp.abs(xs[:, :, None] - xs[:, None, :])
    inter = jnp.maximum(4.0 - dy, 0.0) * jnp.maximum(4.0 - dx, 0.0)
    ov = inter >= 3.0
    sj = scores[:, :, None]
    si = scores[:, None, :]
    prec = (sj > si) | ((sj == si) & (idx[:, :, None] < idx[:, None, :]))
    O = (ov & prec).astype(jnp.float32)
    P = prec.astype(jnp.float32)
    valid = (scores >= DET_THRESH).astype(jnp.float32)

    def round_body(state):
        kept, undec, _ = state
        a_k = jnp.einsum('bj,bji->bi', kept, O)
        a_u = jnp.einsum('bj,bji->bi', undec, O)
        newly_sup = undec * (a_k > 0)
        newly_kept = undec * (a_k == 0) * (a_u == 0)
        kept = kept + newly_kept
        undec = undec - newly_sup - newly_kept
        return kept, undec, jnp.sum(undec)

    def cond(state):
        return state[2] > 0

    kept0 = jnp.zeros((B, K_CAND), jnp.float32)
    kept, _, _ = jax.lax.while_loop(
        cond, round_body, (kept0, valid, jnp.float32(1.0)))
    kr = jnp.einsum('bj,bji->bi', kept, P)
    final = kept * (kr < TOP_K)
    out_vals = jnp.where(final > 0, scores, 0.0)
    gidx = (jnp.arange(B)[:, None] * (HEAT * HEAT) + idx).reshape(-1)
    out = jnp.zeros((B * HEAT * HEAT,), jnp.float32).at[gidx].add(
        out_vals.reshape(-1))
    heat_nms = out.reshape(B, HEAT, HEAT)
    pred = (heat_nms >= DET_THRESH).astype(jnp.int32)
    return heat_nms, pred


@jax.jit
def kernel(x, Wa, ba, Wb, bb):
    B = x.shape[0]
    xt = jnp.transpose(x, (0, 2, 3, 1))
    xp = jnp.pad(xt, ((0, 0), (1, 7), (1, 7), (0, 0)))
    wa = jnp.transpose(Wa, (2, 3, 1, 0)).reshape(9, CIN, CMID)
    wb = jnp.transpose(Wb.reshape(COUT, CMID), (1, 0))
    logits_f, prob_f = _conv_softmax(xp, wa, ba.reshape(1, CMID),
                                     wb, bb.reshape(1, COUT))
    logits = jnp.transpose(logits_f.reshape(B, H, W, COUT), (0, 3, 1, 2))
    heat = jnp.transpose(
        prob_f.reshape(B, H, W, GRID, GRID),
        (0, 1, 3, 2, 4)).reshape(B, HEAT, HEAT)
    heat_nms, pred = _nms_tail(heat.reshape(B, HEAT * HEAT))
    return logits, heat, heat_nms, pred
